# baseline (speedup 1.0000x reference)
import jax
import jax.numpy as jnp
from jax import lax
from jax.experimental import pallas as pl
from jax.experimental.pallas import tpu as pltpu
import numpy as np

B = 64
S = 8
V = 100000
VB = 2560
NBLK = (V + VB - 1) // VB
LCH = VB // 128

_KS0 = np.uint32(0)
_KS1 = np.uint32(42)
_KS2 = np.uint32(42) ^ np.uint32(0x1BD11BDA)
_R0 = (13, 15, 26, 6)
_R1 = (17, 29, 16, 24)
_NEG_INF = np.float32(-np.inf)
_IMAX = np.int32(2**31 - 1)


def _threefry_bits(cnt):
    x0 = jnp.zeros_like(cnt)
    x1 = cnt + _KS1

    def rnd(x0, x1, r):
        x0 = x0 + x1
        x1 = (x1 << np.uint32(r)) | (x1 >> np.uint32(32 - r))
        return x0, x1 ^ x0

    sched = ((_R0, _KS1, _KS2, 1), (_R1, _KS2, _KS0, 2), (_R0, _KS0, _KS1, 3),
             (_R1, _KS1, _KS2, 4), (_R0, _KS2, _KS0, 5))
    for rots, a0, a1, c in sched:
        for r in rots:
            x0, x1 = rnd(x0, x1, r)
        x0 = x0 + a0
        x1 = x1 + (a1 + np.uint32(c))
    return x0 ^ x1


def _gumbel_from_bits(bits):
    fb = (bits >> np.uint32(9)) | np.uint32(0x3F800000)
    u = lax.bitcast_convert_type(fb, jnp.float32) - jnp.float32(1.0)
    return -jnp.log(-jnp.log(u))


def _compose_kernel(x_ref, mask_ref, final_ref, ids_ref, ay_scr, ac_scr):
    j = pl.program_id(0)

    @pl.when(j == 0)
    def _():
        ay_scr[...] = jnp.full((B, 128), _NEG_INF, jnp.float32)
        ac_scr[...] = jnp.zeros((B, 128), jnp.int32)

    lane = lax.broadcasted_iota(jnp.int32, (B, 128), 1)
    row = lax.broadcasted_iota(jnp.int32, (B, 128), 0)
    cnt0 = (row * V + lane + j * VB).astype(jnp.uint32)

    def body(l, carry):
        cnt, acc_y, acc_c = carry
        cnt_next = cnt + np.uint32(128)
        sl = pl.ds(l * 128, 128)
        x = x_ref[:, S - 1, sl]
        mb = mask_ref[:, sl]
        final_ref[:, sl] = x + mb
        g = _gumbel_from_bits(_threefry_bits(cnt))
        y = final_ref[:, sl] + g
        upd = y > acc_y
        ci = j * LCH + l
        return (cnt_next,
                jnp.where(upd, y, acc_y),
                jnp.where(upd, ci, acc_c))

    _, acc_y, acc_c = lax.fori_loop(
        0, LCH, body, (cnt0, ay_scr[...], ac_scr[...]), unroll=20)
    ay_scr[...] = acc_y
    ac_scr[...] = acc_c

    @pl.when(j == NBLK - 1)
    def _():
        m = jnp.max(acc_y, axis=1)
        col = acc_c * 128 + lane
        cand = jnp.where(acc_y == m[:, None], col, _IMAX)
        ids_ref[...] = jnp.min(cand, axis=1)[:, None]


def kernel(logits, prediction_mask):
    mask2 = jnp.pad(prediction_mask, (0, NBLK * VB - V),
                    constant_values=-np.inf)[None, :]
    final, ids2d = pl.pallas_call(
        _compose_kernel,
        grid=(NBLK,),
        in_specs=[
            pl.BlockSpec((B, S, VB), lambda j: (0, 0, j)),
            pl.BlockSpec((1, VB), lambda j: (0, j)),
        ],
        out_specs=[
            pl.BlockSpec((B, VB), lambda j: (0, j)),
            pl.BlockSpec((B, 1), lambda j: (0, 0)),
        ],
        out_shape=[
            jax.ShapeDtypeStruct((B, V), jnp.float32),
            jax.ShapeDtypeStruct((B, 1), jnp.int32),
        ],
        scratch_shapes=[
            pltpu.VMEM((B, 128), jnp.float32),
            pltpu.VMEM((B, 128), jnp.int32),
        ],
    )(logits, mask2)
    return ids2d[:, 0], final


# speedup vs baseline: 1.0156x; 1.0156x over previous
import jax
import jax.numpy as jnp
from jax import lax
from jax.experimental import pallas as pl
from jax.experimental.pallas import tpu as pltpu
import numpy as np

B = 64
S = 8
V = 100000
VB = 3200
NBLK = (V + VB - 1) // VB
LCH = VB // 128

_KS0 = np.uint32(0)
_KS1 = np.uint32(42)
_KS2 = np.uint32(42) ^ np.uint32(0x1BD11BDA)
_R0 = (13, 15, 26, 6)
_R1 = (17, 29, 16, 24)
_NEG_INF = np.float32(-np.inf)
_IMAX = np.int32(2**31 - 1)


def _threefry_bits(cnt):
    x0 = jnp.zeros_like(cnt)
    x1 = cnt + _KS1

    def rnd(x0, x1, r):
        x0 = x0 + x1
        x1 = (x1 << np.uint32(r)) | (x1 >> np.uint32(32 - r))
        return x0, x1 ^ x0

    sched = ((_R0, _KS1, _KS2, 1), (_R1, _KS2, _KS0, 2), (_R0, _KS0, _KS1, 3),
             (_R1, _KS1, _KS2, 4), (_R0, _KS2, _KS0, 5))
    for rots, a0, a1, c in sched:
        for r in rots:
            x0, x1 = rnd(x0, x1, r)
        x0 = x0 + a0
        x1 = x1 + (a1 + np.uint32(c))
    return x0 ^ x1


def _gumbel_from_bits(bits):
    fb = (bits >> np.uint32(9)) | np.uint32(0x3F800000)
    u = lax.bitcast_convert_type(fb, jnp.float32) - jnp.float32(1.0)
    return -jnp.log(-jnp.log(u))


def _compose_kernel(x_ref, mask_ref, final_ref, ids_ref, ay_scr, ac_scr):
    j = pl.program_id(0)

    @pl.when(j == 0)
    def _():
        ay_scr[...] = jnp.full((B, 128), _NEG_INF, jnp.float32)
        ac_scr[...] = jnp.zeros((B, 128), jnp.int32)

    lane = lax.broadcasted_iota(jnp.int32, (B, 128), 1)
    row = lax.broadcasted_iota(jnp.int32, (B, 128), 0)
    cnt0 = (row * V + lane + j * VB).astype(jnp.uint32)

    def body(l, carry):
        cnt, acc_y, acc_c = carry
        cnt_next = cnt + np.uint32(128)
        sl = pl.ds(l * 128, 128)
        x = x_ref[:, S - 1, sl]
        mb = mask_ref[:, sl]
        final_ref[:, sl] = x + mb
        g = _gumbel_from_bits(_threefry_bits(cnt))
        y = final_ref[:, sl] + g
        upd = y > acc_y
        ci = j * LCH + l
        return (cnt_next,
                jnp.where(upd, y, acc_y),
                jnp.where(upd, ci, acc_c))

    _, acc_y, acc_c = lax.fori_loop(
        0, LCH, body, (cnt0, ay_scr[...], ac_scr[...]), unroll=25)
    ay_scr[...] = acc_y
    ac_scr[...] = acc_c

    @pl.when(j == NBLK - 1)
    def _():
        m = jnp.max(acc_y, axis=1)
        col = acc_c * 128 + lane
        cand = jnp.where(acc_y == m[:, None], col, _IMAX)
        ids_ref[...] = jnp.min(cand, axis=1)[:, None]


def kernel(logits, prediction_mask):
    mask2 = jnp.pad(prediction_mask, (0, NBLK * VB - V),
                    constant_values=-np.inf)[None, :]
    final, ids2d = pl.pallas_call(
        _compose_kernel,
        grid=(NBLK,),
        in_specs=[
            pl.BlockSpec((B, S, VB), lambda j: (0, 0, j)),
            pl.BlockSpec((1, VB), lambda j: (0, j)),
        ],
        out_specs=[
            pl.BlockSpec((B, VB), lambda j: (0, j)),
            pl.BlockSpec((B, 1), lambda j: (0, 0)),
        ],
        out_shape=[
            jax.ShapeDtypeStruct((B, V), jnp.float32),
            jax.ShapeDtypeStruct((B, 1), jnp.int32),
        ],
        scratch_shapes=[
            pltpu.VMEM((B, 128), jnp.float32),
            pltpu.VMEM((B, 128), jnp.int32),
        ],
    )(logits, mask2)
    return ids2d[:, 0], final
